# baseline (speedup 1.0000x reference)
import functools

import jax
import jax.numpy as jnp
from jax import lax
from jax.experimental import pallas as pl
from jax.experimental.pallas import tpu as pltpu
from jax.experimental.pallas import tpu_sc as plsc

N = 50000
E = 100000
B = 1024
H = 32

NPAD = 50176
EPAD = 102400
NW = 32
NCHUNK = 25
CW = 128
MOLR = 1280

_SC_PARAMS = pltpu.CompilerParams(use_tc_tiling_on_sc=False)

TN = 512
TE = 256


def _h0_body(nf_ref, w_ref, b_ref, o_ref):
    o_ref[...] = jax.nn.relu(
        jnp.dot(nf_ref[...], w_ref[...], preferred_element_type=jnp.float32)
        + b_ref[...])


def _tc_h0(nf, W_proj, b_proj):
    return pl.pallas_call(
        _h0_body,
        grid=(NPAD // TN,),
        in_specs=[
            pl.BlockSpec((TN, nf.shape[1]), lambda i: (i, 0)),
            pl.BlockSpec((nf.shape[1], H), lambda i: (0, 0)),
            pl.BlockSpec((1, H), lambda i: (0, 0)),
        ],
        out_specs=pl.BlockSpec((TN, H), lambda i: (i, 0)),
        out_shape=jax.ShapeDtypeStruct((NPAD, H), jnp.float32),
    )(nf, W_proj, b_proj)


def _edge_emb_body(ef_ref, w_ref, b_ref, o_ref):
    o_ref[...] = jax.nn.relu(
        jnp.dot(ef_ref[...], w_ref[...], preferred_element_type=jnp.float32)
        + b_ref[...])


def _tc_edge_emb(ef8, W_pe8, b_pe):
    return pl.pallas_call(
        _edge_emb_body,
        grid=(EPAD // TN,),
        in_specs=[
            pl.BlockSpec((TN, 8), lambda i: (i, 0)),
            pl.BlockSpec((8, 32), lambda i: (0, 0)),
            pl.BlockSpec((1, 32), lambda i: (0, 0)),
        ],
        out_specs=pl.BlockSpec((TN, 32), lambda i: (i, 0)),
        out_shape=jax.ShapeDtypeStruct((EPAD, 32), jnp.float32),
    )(ef8, W_pe8, b_pe)


def _edge_body(ef_ref, hs_ref, w1_ref, b1_ref, w2_ref, b2_ref, o_ref):
    e1 = jax.nn.relu(
        jnp.dot(ef_ref[...], w1_ref[...], preferred_element_type=jnp.float32)
        + b1_ref[...])
    A = jnp.dot(e1, w2_ref[...], preferred_element_type=jnp.float32) + b2_ref[...]
    A3 = A.reshape(TE, H, H)
    hs = hs_ref[...]
    o_ref[...] = jnp.sum(A3 * hs[:, :, None], axis=1)


def _tc_edge(ef8, hs, W_e1p, b_e1, W_e2, b_e2):
    return pl.pallas_call(
        _edge_body,
        grid=(EPAD // TE,),
        in_specs=[
            pl.BlockSpec((TE, 8), lambda i: (i, 0)),
            pl.BlockSpec((TE, H), lambda i: (i, 0)),
            pl.BlockSpec((8, 128), lambda i: (0, 0)),
            pl.BlockSpec((1, 128), lambda i: (0, 0)),
            pl.BlockSpec((128, H * H), lambda i: (0, 0)),
            pl.BlockSpec((1, H * H), lambda i: (0, 0)),
        ],
        out_specs=pl.BlockSpec((TE, H), lambda i: (i, 0)),
        out_shape=jax.ShapeDtypeStruct((EPAD, H), jnp.float32),
    )(ef8, hs, W_e1p, b_e1, W_e2, b_e2)


def _gru_body(a0_ref, a1_ref, h_ref, wir, wiz, win, whr, whz, whn, bi, bh, o_ref):
    h = h_ref[...]
    m = jax.nn.relu(a0_ref[...] + a1_ref[...]) + h

    def mm(x, w):
        return jnp.dot(x, w[...], preferred_element_type=jnp.float32)

    r = jax.nn.sigmoid(mm(m, wir) + bi[:, 0:H] + mm(h, whr) + bh[:, 0:H])
    z = jax.nn.sigmoid(mm(m, wiz) + bi[:, H:2 * H] + mm(h, whz) + bh[:, H:2 * H])
    n = jnp.tanh(mm(m, win) + bi[:, 2 * H:] + r * (mm(h, whn) + bh[:, 2 * H:]))
    o_ref[...] = (1.0 - z) * n + z * h


def _tc_gru(a0, a1, h, Wsplits, b_ih, b_hh):
    wir, wiz, win, whr, whz, whn = Wsplits
    specs = [pl.BlockSpec((TN, H), lambda i: (i, 0))] * 3
    specs += [pl.BlockSpec((H, H), lambda i: (0, 0))] * 6
    specs += [pl.BlockSpec((1, 3 * H), lambda i: (0, 0))] * 2
    return pl.pallas_call(
        _gru_body,
        grid=(NPAD // TN,),
        in_specs=specs,
        out_specs=pl.BlockSpec((TN, H), lambda i: (i, 0)),
        out_shape=jax.ShapeDtypeStruct((NPAD, H), jnp.float32),
    )(a0, a1, h, wir, wiz, win, whr, whz, whn, b_ih, b_hh)


def _final_body(molA_ref, molB_ref, fp_ref, wfp_ref, bfp_ref, g_ref, be_ref,
                wf1a_ref, wf1b_ref, wf1c_ref, bf1_ref, wf2_ref, bf2_ref,
                wh_ref, bh_ref, o_ref):
    a = molA_ref[0, 0:B, :] + molA_ref[1, 0:B, :]
    b = molB_ref[0, 0:B, :] + molB_ref[1, 0:B, :]
    m = jnp.maximum(jnp.max(a, axis=1, keepdims=True),
                    jnp.max(b, axis=1, keepdims=True))
    ea = jnp.exp(a - m)
    eb = jnp.exp(b - m)
    s = jnp.sum(ea, axis=1, keepdims=True) + jnp.sum(eb, axis=1, keepdims=True)
    an = ea / s
    bn = eb / s
    x = jnp.dot(fp_ref[...], wfp_ref[...], preferred_element_type=jnp.float32) \
        + bfp_ref[...]
    mean = jnp.mean(x, axis=0, keepdims=True)
    var = jnp.mean(x * x, axis=0, keepdims=True) - mean * mean
    x = (x - mean) * jax.lax.rsqrt(var + 1e-5)
    x = jax.nn.relu(x * g_ref[...] + be_ref[...])

    def mm(p, w):
        return jnp.dot(p, w[...], preferred_element_type=jnp.float32)

    h1 = jax.nn.relu(mm(an, wf1a_ref) + mm(bn, wf1b_ref) + mm(x, wf1c_ref)
                     + bf1_ref[...])
    emb = mm(h1, wf2_ref) + bf2_ref[...]
    o_ref[...] = mm(emb, wh_ref) + bh_ref[...]


def _tc_final(molA2, molB2, fp, W_fp, b_fp, gam, bet,
              W_f1a, W_f1b, W_f1c, b_f1, W_f2, b_f2, W_head, b_head):
    args = (molA2, molB2, fp, W_fp, b_fp, gam, bet,
            W_f1a, W_f1b, W_f1c, b_f1, W_f2, b_f2, W_head, b_head)
    return pl.pallas_call(
        _final_body,
        in_specs=[pl.BlockSpec(a.shape, functools.partial(lambda r: (0,) * r, len(a.shape)))
                  for a in args],
        out_specs=pl.BlockSpec((B, W_head.shape[1]), lambda: (0, 0)),
        out_shape=jax.ShapeDtypeStruct((B, W_head.shape[1]), jnp.float32),
    )(*args)


def _sc_gather(h, idx3):
    mesh = plsc.VectorSubcoreMesh(core_axis_name="c", subcore_axis_name="s")

    @functools.partial(
        pl.kernel, mesh=mesh,
        out_type=jax.ShapeDtypeStruct((EPAD, H), jnp.float32),
        scratch_types=[
            pltpu.VMEM((NCHUNK, CW), jnp.int32),
            pltpu.VMEM((CW, H), jnp.float32),
            pltpu.SemaphoreType.DMA,
        ],
        compiler_params=_SC_PARAMS,
    )
    def k(h_hbm, idx_hbm, out_hbm, idx_v, rows_v, sem):
        c = lax.axis_index("c")
        s = lax.axis_index("s")
        wid = c * 16 + s
        pltpu.sync_copy(idx_hbm.at[wid], idx_v)

        def body(i, _):
            pltpu.async_copy(h_hbm.at[idx_v.at[i]], rows_v, sem).wait()
            pltpu.sync_copy(rows_v, out_hbm.at[pl.ds(wid * NCHUNK * CW + i * CW, CW)])
            return ()

        lax.fori_loop(0, NCHUNK, body, (), unroll=False)

    return k(h, idx3)


def _sc_scatter(msg, dst3, zeros_n):
    mesh = plsc.VectorSubcoreMesh(core_axis_name="c", subcore_axis_name="s")
    rows_per_tile = NPAD // 16

    @functools.partial(
        pl.kernel, mesh=mesh,
        out_type=jax.ShapeDtypeStruct((2, NPAD, H), jnp.float32),
        scratch_types=[
            pltpu.VMEM((NCHUNK, CW), jnp.int32),
            pltpu.VMEM((CW, H), jnp.float32),
            pltpu.VMEM_SHARED((NPAD, H), jnp.float32),
            pltpu.SemaphoreType.DMA,
        ],
        compiler_params=_SC_PARAMS,
    )
    def k(msg_hbm, idx_hbm, zero_hbm, out_hbm, idx_v, rows_v, acc, sem):
        c = lax.axis_index("c")
        s = lax.axis_index("s")
        wid = c * 16 + s
        pltpu.sync_copy(zero_hbm.at[pl.ds(0, rows_per_tile)],
                        acc.at[pl.ds(s * rows_per_tile, rows_per_tile)])
        pltpu.sync_copy(idx_hbm.at[wid], idx_v)
        plsc.subcore_barrier()

        def body(i, _):
            pltpu.async_copy(
                msg_hbm.at[pl.ds(wid * NCHUNK * CW + i * CW, CW)], rows_v, sem
            ).wait()
            pltpu.sync_copy(rows_v, acc.at[idx_v.at[i]], add=True)
            return ()

        lax.fori_loop(0, NCHUNK, body, (), unroll=False)
        plsc.subcore_barrier()
        pltpu.sync_copy(acc.at[pl.ds(s * rows_per_tile, rows_per_tile)],
                        out_hbm.at[c].at[pl.ds(s * rows_per_tile, rows_per_tile)])

    return k(msg, dst3, zeros_n)


def _sc_gid(dst3, ngi):
    mesh = plsc.VectorSubcoreMesh(core_axis_name="c", subcore_axis_name="s")
    rows_per_tile = NPAD // 16

    @functools.partial(
        pl.kernel, mesh=mesh,
        out_type=jax.ShapeDtypeStruct((NW, NCHUNK, CW), jnp.int32),
        scratch_types=[
            pltpu.VMEM((NCHUNK, CW), jnp.int32),
            pltpu.VMEM((NCHUNK, CW), jnp.int32),
            pltpu.VMEM((NPAD // 16,), jnp.int32),
            pltpu.VMEM_SHARED((NPAD,), jnp.int32),
            pltpu.SemaphoreType.DMA,
        ],
        compiler_params=_SC_PARAMS,
    )
    def k(dst_hbm, ngi_hbm, out_hbm, dst_v, gid_v, ngi_stage, ngi_sp, sem):
        c = lax.axis_index("c")
        s = lax.axis_index("s")
        wid = c * 16 + s
        pltpu.sync_copy(ngi_hbm.at[pl.ds(s * rows_per_tile, rows_per_tile)],
                        ngi_stage)
        pltpu.sync_copy(ngi_stage,
                        ngi_sp.at[pl.ds(s * rows_per_tile, rows_per_tile)])
        pltpu.sync_copy(dst_hbm.at[wid], dst_v)
        plsc.subcore_barrier()

        def body(i, _):
            pltpu.async_copy(ngi_sp.at[dst_v.at[i]], gid_v.at[i], sem).wait()
            return ()

        lax.fori_loop(0, NCHUNK, body, (), unroll=False)
        pltpu.sync_copy(gid_v, out_hbm.at[wid])

    return k(dst3, ngi)


def _sc_scatter_mol(msgA, msgB, gid3, zeros_n):
    mesh = plsc.VectorSubcoreMesh(core_axis_name="c", subcore_axis_name="s")
    mol_per_tile = MOLR // 16

    @functools.partial(
        pl.kernel, mesh=mesh,
        out_type=(jax.ShapeDtypeStruct((2, MOLR, H), jnp.float32),
                  jax.ShapeDtypeStruct((2, MOLR, H), jnp.float32)),
        scratch_types=[
            pltpu.VMEM((NCHUNK, CW), jnp.int32),
            pltpu.VMEM((CW, H), jnp.float32),
            pltpu.VMEM((CW, H), jnp.float32),
            pltpu.VMEM_SHARED((MOLR, H), jnp.float32),
            pltpu.VMEM_SHARED((MOLR, H), jnp.float32),
            pltpu.SemaphoreType.DMA,
        ],
        compiler_params=_SC_PARAMS,
    )
    def k(msgA_hbm, msgB_hbm, gid_hbm, zero_hbm, outA_hbm, outB_hbm,
          gid_v, arows_v, brows_v, accA, accB, sem):
        c = lax.axis_index("c")
        s = lax.axis_index("s")
        wid = c * 16 + s
        pltpu.sync_copy(zero_hbm.at[pl.ds(0, mol_per_tile)],
                        accA.at[pl.ds(s * mol_per_tile, mol_per_tile)])
        pltpu.sync_copy(zero_hbm.at[pl.ds(0, mol_per_tile)],
                        accB.at[pl.ds(s * mol_per_tile, mol_per_tile)])
        pltpu.sync_copy(gid_hbm.at[wid], gid_v)
        plsc.subcore_barrier()

        def body(i, _):
            base = wid * NCHUNK * CW + i * CW
            pltpu.async_copy(msgA_hbm.at[pl.ds(base, CW)], arows_v, sem).wait()
            pltpu.async_copy(msgB_hbm.at[pl.ds(base, CW)], brows_v, sem).wait()
            pltpu.sync_copy(arows_v, accA.at[gid_v.at[i]], add=True)
            pltpu.sync_copy(brows_v, accB.at[gid_v.at[i]], add=True)
            return ()

        lax.fori_loop(0, NCHUNK, body, (), unroll=False)
        plsc.subcore_barrier()
        pltpu.sync_copy(accA.at[pl.ds(s * mol_per_tile, mol_per_tile)],
                        outA_hbm.at[c].at[pl.ds(s * mol_per_tile, mol_per_tile)])
        pltpu.sync_copy(accB.at[pl.ds(s * mol_per_tile, mol_per_tile)],
                        outB_hbm.at[c].at[pl.ds(s * mol_per_tile, mol_per_tile)])

    return k(msgA, msgB, gid3, zeros_n)


def kernel(node_feats, edge_feats, fp_vector, edge_index, node_graph_ids,
           W_proj, b_proj, W_e1, b_e1, W_e2, b_e2,
           W_ih, W_hh, b_ih, b_hh, W_pe, b_pe,
           W_fp, b_fp, bn_gamma, bn_beta,
           W_f1, b_f1, W_f2, b_f2, W_head, b_head):
    f32 = jnp.float32
    nf = jnp.pad(node_feats, ((0, NPAD - N), (0, 0)))
    ef8 = jnp.pad(edge_feats, ((0, EPAD - E), (0, 8 - edge_feats.shape[1])))
    pad_src = jnp.arange(EPAD - E, dtype=jnp.int32) % 128
    pad_dst = N + jnp.arange(EPAD - E, dtype=jnp.int32) % (NPAD - N)
    src = jnp.concatenate([edge_index[0].astype(jnp.int32), pad_src])
    dst = jnp.concatenate([edge_index[1].astype(jnp.int32), pad_dst])
    src3 = src.reshape(NW, NCHUNK, CW)
    dst3 = dst.reshape(NW, NCHUNK, CW)
    ngi = jnp.concatenate([
        node_graph_ids.astype(jnp.int32),
        B + jnp.arange(NPAD - N, dtype=jnp.int32) % (MOLR - B)])
    zeros_n = jnp.zeros((NPAD, H), f32)

    W_e1p = jnp.pad(W_e1, ((0, 8 - W_e1.shape[0]), (0, 0)))
    W_pe8 = jnp.pad(W_pe, ((0, 8 - W_pe.shape[0]), (0, 0)))
    b_e1r = b_e1.reshape(1, -1)
    b_e2r = b_e2.reshape(1, -1)
    b_per = b_pe.reshape(1, -1)
    b_projr = b_proj.reshape(1, -1)
    Wsplits = (W_ih[:, 0:H], W_ih[:, H:2 * H], W_ih[:, 2 * H:],
               W_hh[:, 0:H], W_hh[:, H:2 * H], W_hh[:, 2 * H:])
    b_ihr = b_ih.reshape(1, -1)
    b_hhr = b_hh.reshape(1, -1)
    W_f1a = W_f1[0:H]
    W_f1b = W_f1[H:2 * H]
    W_f1c = W_f1[2 * H:]

    h = _tc_h0(nf, W_proj, b_projr)
    edge_emb = _tc_edge_emb(ef8, W_pe8, b_per)
    for _ in range(3):
        hs = _sc_gather(h, src3)
        msg = _tc_edge(ef8, hs, W_e1p, b_e1r, W_e2, b_e2r)
        agg2 = _sc_scatter(msg, dst3, zeros_n)
        h = _tc_gru(agg2[0], agg2[1], h, Wsplits, b_ihr, b_hhr)
    hs_f = _sc_gather(h, src3)
    gid3 = _sc_gid(dst3, ngi)
    molA2, molB2 = _sc_scatter_mol(hs_f, edge_emb, gid3, zeros_n)
    return _tc_final(molA2, molB2, fp_vector, W_fp, b_fp,
                     bn_gamma.reshape(1, -1), bn_beta.reshape(1, -1),
                     W_f1a, W_f1b, W_f1c, b_f1.reshape(1, -1),
                     W_f2, b_f2.reshape(1, -1), W_head, b_head.reshape(1, -1))


# speedup vs baseline: 1.5972x; 1.5972x over previous
import functools

import jax
import jax.numpy as jnp
from jax import lax
from jax.experimental import pallas as pl
from jax.experimental.pallas import tpu as pltpu
from jax.experimental.pallas import tpu_sc as plsc

N = 50000
E = 100000
B = 1024
H = 32

NPAD = 50176
EPAD = 102400
NW = 32
NCHUNK = 25
CW = 128
MOLR = 1280

_SC_PARAMS = pltpu.CompilerParams(use_tc_tiling_on_sc=False)

TN = 512
TE = 256


def _h0_body(nf_ref, w_ref, b_ref, o_ref):
    o_ref[...] = jax.nn.relu(
        jnp.dot(nf_ref[...], w_ref[...], preferred_element_type=jnp.float32)
        + b_ref[...])


def _tc_h0(nf, W_proj, b_proj):
    return pl.pallas_call(
        _h0_body,
        grid=(NPAD // TN,),
        in_specs=[
            pl.BlockSpec((TN, nf.shape[1]), lambda i: (i, 0)),
            pl.BlockSpec((nf.shape[1], H), lambda i: (0, 0)),
            pl.BlockSpec((1, H), lambda i: (0, 0)),
        ],
        out_specs=pl.BlockSpec((TN, H), lambda i: (i, 0)),
        out_shape=jax.ShapeDtypeStruct((NPAD, H), jnp.float32),
    )(nf, W_proj, b_proj)


def _edge_emb_body(ef_ref, w_ref, b_ref, o_ref):
    o_ref[...] = jax.nn.relu(
        jnp.dot(ef_ref[...], w_ref[...], preferred_element_type=jnp.float32)
        + b_ref[...])


def _tc_edge_emb(ef8, W_pe8, b_pe):
    return pl.pallas_call(
        _edge_emb_body,
        grid=(EPAD // TN,),
        in_specs=[
            pl.BlockSpec((TN, 8), lambda i: (i, 0)),
            pl.BlockSpec((8, 32), lambda i: (0, 0)),
            pl.BlockSpec((1, 32), lambda i: (0, 0)),
        ],
        out_specs=pl.BlockSpec((TN, 32), lambda i: (i, 0)),
        out_shape=jax.ShapeDtypeStruct((EPAD, 32), jnp.float32),
    )(ef8, W_pe8, b_pe)


def _edge_body(ef_ref, hs_ref, w1_ref, b1_ref, w2_ref, r_ref, s_ref, b2m_ref,
               o_ref):
    e1 = jax.nn.relu(
        jnp.dot(ef_ref[...], w1_ref[...], preferred_element_type=jnp.float32)
        + b1_ref[...])
    A = jnp.dot(e1, w2_ref[...], preferred_element_type=jnp.float32)
    hs = hs_ref[...]
    hrep = jnp.dot(hs, r_ref[...], preferred_element_type=jnp.float32)
    o_ref[...] = (
        jnp.dot(hrep * A, s_ref[...], preferred_element_type=jnp.float32)
        + jnp.dot(hs, b2m_ref[...], preferred_element_type=jnp.float32))


def _tc_edge(ef8, hs, W_e1p, b_e1, W_e2, Rm, Sm, B2m):
    return pl.pallas_call(
        _edge_body,
        grid=(EPAD // TE,),
        in_specs=[
            pl.BlockSpec((TE, 8), lambda i: (i, 0)),
            pl.BlockSpec((TE, H), lambda i: (i, 0)),
            pl.BlockSpec((8, 128), lambda i: (0, 0)),
            pl.BlockSpec((1, 128), lambda i: (0, 0)),
            pl.BlockSpec((128, H * H), lambda i: (0, 0)),
            pl.BlockSpec((H, H * H), lambda i: (0, 0)),
            pl.BlockSpec((H * H, H), lambda i: (0, 0)),
            pl.BlockSpec((H, H), lambda i: (0, 0)),
        ],
        out_specs=pl.BlockSpec((TE, H), lambda i: (i, 0)),
        out_shape=jax.ShapeDtypeStruct((EPAD, H), jnp.float32),
    )(ef8, hs, W_e1p, b_e1, W_e2, Rm, Sm, B2m)


def _gru_body(a0_ref, a1_ref, h_ref, wir, wiz, win, whr, whz, whn, bi, bh, o_ref):
    h = h_ref[...]
    m = jax.nn.relu(a0_ref[...] + a1_ref[...]) + h

    def mm(x, w):
        return jnp.dot(x, w[...], preferred_element_type=jnp.float32)

    r = jax.nn.sigmoid(mm(m, wir) + bi[:, 0:H] + mm(h, whr) + bh[:, 0:H])
    z = jax.nn.sigmoid(mm(m, wiz) + bi[:, H:2 * H] + mm(h, whz) + bh[:, H:2 * H])
    n = jnp.tanh(mm(m, win) + bi[:, 2 * H:] + r * (mm(h, whn) + bh[:, 2 * H:]))
    o_ref[...] = (1.0 - z) * n + z * h


def _tc_gru(a0, a1, h, Wsplits, b_ih, b_hh):
    wir, wiz, win, whr, whz, whn = Wsplits
    specs = [pl.BlockSpec((TN, H), lambda i: (i, 0))] * 3
    specs += [pl.BlockSpec((H, H), lambda i: (0, 0))] * 6
    specs += [pl.BlockSpec((1, 3 * H), lambda i: (0, 0))] * 2
    return pl.pallas_call(
        _gru_body,
        grid=(NPAD // TN,),
        in_specs=specs,
        out_specs=pl.BlockSpec((TN, H), lambda i: (i, 0)),
        out_shape=jax.ShapeDtypeStruct((NPAD, H), jnp.float32),
    )(a0, a1, h, wir, wiz, win, whr, whz, whn, b_ih, b_hh)


def _final_body(molA_ref, molB_ref, fp_ref, wfp_ref, bfp_ref, g_ref, be_ref,
                wf1a_ref, wf1b_ref, wf1c_ref, bf1_ref, wf2_ref, bf2_ref,
                wh_ref, bh_ref, o_ref):
    a = molA_ref[0, 0:B, :] + molA_ref[1, 0:B, :]
    b = molB_ref[0, 0:B, :] + molB_ref[1, 0:B, :]
    m = jnp.maximum(jnp.max(a, axis=1, keepdims=True),
                    jnp.max(b, axis=1, keepdims=True))
    ea = jnp.exp(a - m)
    eb = jnp.exp(b - m)
    s = jnp.sum(ea, axis=1, keepdims=True) + jnp.sum(eb, axis=1, keepdims=True)
    an = ea / s
    bn = eb / s
    x = jnp.dot(fp_ref[...], wfp_ref[...], preferred_element_type=jnp.float32) \
        + bfp_ref[...]
    mean = jnp.mean(x, axis=0, keepdims=True)
    var = jnp.mean(x * x, axis=0, keepdims=True) - mean * mean
    x = (x - mean) * jax.lax.rsqrt(var + 1e-5)
    x = jax.nn.relu(x * g_ref[...] + be_ref[...])

    def mm(p, w):
        return jnp.dot(p, w[...], preferred_element_type=jnp.float32)

    h1 = jax.nn.relu(mm(an, wf1a_ref) + mm(bn, wf1b_ref) + mm(x, wf1c_ref)
                     + bf1_ref[...])
    emb = mm(h1, wf2_ref) + bf2_ref[...]
    o_ref[...] = mm(emb, wh_ref) + bh_ref[...]


def _tc_final(molA2, molB2, fp, W_fp, b_fp, gam, bet,
              W_f1a, W_f1b, W_f1c, b_f1, W_f2, b_f2, W_head, b_head):
    args = (molA2, molB2, fp, W_fp, b_fp, gam, bet,
            W_f1a, W_f1b, W_f1c, b_f1, W_f2, b_f2, W_head, b_head)
    return pl.pallas_call(
        _final_body,
        in_specs=[pl.BlockSpec(a.shape, functools.partial(lambda r: (0,) * r, len(a.shape)))
                  for a in args],
        out_specs=pl.BlockSpec((B, W_head.shape[1]), lambda: (0, 0)),
        out_shape=jax.ShapeDtypeStruct((B, W_head.shape[1]), jnp.float32),
    )(*args)


def _sc_gather(h, idx3):
    mesh = plsc.VectorSubcoreMesh(core_axis_name="c", subcore_axis_name="s")

    @functools.partial(
        pl.kernel, mesh=mesh,
        out_type=jax.ShapeDtypeStruct((EPAD, H), jnp.float32),
        scratch_types=[
            pltpu.VMEM((NCHUNK, CW), jnp.int32),
            pltpu.VMEM((CW, H), jnp.float32),
            pltpu.SemaphoreType.DMA,
        ],
        compiler_params=_SC_PARAMS,
    )
    def k(h_hbm, idx_hbm, out_hbm, idx_v, rows_v, sem):
        c = lax.axis_index("c")
        s = lax.axis_index("s")
        wid = c * 16 + s
        pltpu.sync_copy(idx_hbm.at[wid], idx_v)

        def body(i, _):
            pltpu.async_copy(h_hbm.at[idx_v.at[i]], rows_v, sem).wait()
            pltpu.sync_copy(rows_v, out_hbm.at[pl.ds(wid * NCHUNK * CW + i * CW, CW)])
            return ()

        lax.fori_loop(0, NCHUNK, body, (), unroll=False)

    return k(h, idx3)


def _sc_scatter(msg, dst3, zeros_n):
    mesh = plsc.VectorSubcoreMesh(core_axis_name="c", subcore_axis_name="s")
    rows_per_tile = NPAD // 16

    @functools.partial(
        pl.kernel, mesh=mesh,
        out_type=jax.ShapeDtypeStruct((2, NPAD, H), jnp.float32),
        scratch_types=[
            pltpu.VMEM((NCHUNK, CW), jnp.int32),
            pltpu.VMEM((CW, H), jnp.float32),
            pltpu.VMEM_SHARED((NPAD, H), jnp.float32),
            pltpu.SemaphoreType.DMA,
        ],
        compiler_params=_SC_PARAMS,
    )
    def k(msg_hbm, idx_hbm, zero_hbm, out_hbm, idx_v, rows_v, acc, sem):
        c = lax.axis_index("c")
        s = lax.axis_index("s")
        wid = c * 16 + s
        pltpu.sync_copy(zero_hbm.at[pl.ds(0, rows_per_tile)],
                        acc.at[pl.ds(s * rows_per_tile, rows_per_tile)])
        pltpu.sync_copy(idx_hbm.at[wid], idx_v)
        plsc.subcore_barrier()

        def body(i, _):
            pltpu.async_copy(
                msg_hbm.at[pl.ds(wid * NCHUNK * CW + i * CW, CW)], rows_v, sem
            ).wait()
            pltpu.sync_copy(rows_v, acc.at[idx_v.at[i]], add=True)
            return ()

        lax.fori_loop(0, NCHUNK, body, (), unroll=False)
        plsc.subcore_barrier()
        pltpu.sync_copy(acc.at[pl.ds(s * rows_per_tile, rows_per_tile)],
                        out_hbm.at[c].at[pl.ds(s * rows_per_tile, rows_per_tile)])

    return k(msg, dst3, zeros_n)


def _sc_gid(dst3, ngi):
    mesh = plsc.VectorSubcoreMesh(core_axis_name="c", subcore_axis_name="s")
    rows_per_tile = NPAD // 16

    @functools.partial(
        pl.kernel, mesh=mesh,
        out_type=jax.ShapeDtypeStruct((NW, NCHUNK, CW), jnp.int32),
        scratch_types=[
            pltpu.VMEM((NCHUNK, CW), jnp.int32),
            pltpu.VMEM((NCHUNK, CW), jnp.int32),
            pltpu.VMEM((NPAD // 16,), jnp.int32),
            pltpu.VMEM_SHARED((NPAD,), jnp.int32),
            pltpu.SemaphoreType.DMA,
        ],
        compiler_params=_SC_PARAMS,
    )
    def k(dst_hbm, ngi_hbm, out_hbm, dst_v, gid_v, ngi_stage, ngi_sp, sem):
        c = lax.axis_index("c")
        s = lax.axis_index("s")
        wid = c * 16 + s
        pltpu.sync_copy(ngi_hbm.at[pl.ds(s * rows_per_tile, rows_per_tile)],
                        ngi_stage)
        pltpu.sync_copy(ngi_stage,
                        ngi_sp.at[pl.ds(s * rows_per_tile, rows_per_tile)])
        pltpu.sync_copy(dst_hbm.at[wid], dst_v)
        plsc.subcore_barrier()

        def body(i, _):
            pltpu.async_copy(ngi_sp.at[dst_v.at[i]], gid_v.at[i], sem).wait()
            return ()

        lax.fori_loop(0, NCHUNK, body, (), unroll=False)
        pltpu.sync_copy(gid_v, out_hbm.at[wid])

    return k(dst3, ngi)


def _sc_scatter_mol(msgA, msgB, gid3, zeros_n):
    mesh = plsc.VectorSubcoreMesh(core_axis_name="c", subcore_axis_name="s")
    mol_per_tile = MOLR // 16

    @functools.partial(
        pl.kernel, mesh=mesh,
        out_type=(jax.ShapeDtypeStruct((2, MOLR, H), jnp.float32),
                  jax.ShapeDtypeStruct((2, MOLR, H), jnp.float32)),
        scratch_types=[
            pltpu.VMEM((NCHUNK, CW), jnp.int32),
            pltpu.VMEM((CW, H), jnp.float32),
            pltpu.VMEM((CW, H), jnp.float32),
            pltpu.VMEM_SHARED((MOLR, H), jnp.float32),
            pltpu.VMEM_SHARED((MOLR, H), jnp.float32),
            pltpu.SemaphoreType.DMA,
        ],
        compiler_params=_SC_PARAMS,
    )
    def k(msgA_hbm, msgB_hbm, gid_hbm, zero_hbm, outA_hbm, outB_hbm,
          gid_v, arows_v, brows_v, accA, accB, sem):
        c = lax.axis_index("c")
        s = lax.axis_index("s")
        wid = c * 16 + s
        pltpu.sync_copy(zero_hbm.at[pl.ds(0, mol_per_tile)],
                        accA.at[pl.ds(s * mol_per_tile, mol_per_tile)])
        pltpu.sync_copy(zero_hbm.at[pl.ds(0, mol_per_tile)],
                        accB.at[pl.ds(s * mol_per_tile, mol_per_tile)])
        pltpu.sync_copy(gid_hbm.at[wid], gid_v)
        plsc.subcore_barrier()

        def body(i, _):
            base = wid * NCHUNK * CW + i * CW
            pltpu.async_copy(msgA_hbm.at[pl.ds(base, CW)], arows_v, sem).wait()
            pltpu.async_copy(msgB_hbm.at[pl.ds(base, CW)], brows_v, sem).wait()
            pltpu.sync_copy(arows_v, accA.at[gid_v.at[i]], add=True)
            pltpu.sync_copy(brows_v, accB.at[gid_v.at[i]], add=True)
            return ()

        lax.fori_loop(0, NCHUNK, body, (), unroll=False)
        plsc.subcore_barrier()
        pltpu.sync_copy(accA.at[pl.ds(s * mol_per_tile, mol_per_tile)],
                        outA_hbm.at[c].at[pl.ds(s * mol_per_tile, mol_per_tile)])
        pltpu.sync_copy(accB.at[pl.ds(s * mol_per_tile, mol_per_tile)],
                        outB_hbm.at[c].at[pl.ds(s * mol_per_tile, mol_per_tile)])

    return k(msgA, msgB, gid3, zeros_n)


def kernel(node_feats, edge_feats, fp_vector, edge_index, node_graph_ids,
           W_proj, b_proj, W_e1, b_e1, W_e2, b_e2,
           W_ih, W_hh, b_ih, b_hh, W_pe, b_pe,
           W_fp, b_fp, bn_gamma, bn_beta,
           W_f1, b_f1, W_f2, b_f2, W_head, b_head):
    f32 = jnp.float32
    nf = jnp.pad(node_feats, ((0, NPAD - N), (0, 0)))
    ef8 = jnp.pad(edge_feats, ((0, EPAD - E), (0, 8 - edge_feats.shape[1])))
    pad_src = jnp.arange(EPAD - E, dtype=jnp.int32) % 128
    pad_dst = N + jnp.arange(EPAD - E, dtype=jnp.int32) % (NPAD - N)
    src = jnp.concatenate([edge_index[0].astype(jnp.int32), pad_src])
    dst = jnp.concatenate([edge_index[1].astype(jnp.int32), pad_dst])
    src3 = src.reshape(NW, NCHUNK, CW)
    dst3 = dst.reshape(NW, NCHUNK, CW)
    ngi = jnp.concatenate([
        node_graph_ids.astype(jnp.int32),
        B + jnp.arange(NPAD - N, dtype=jnp.int32) % (MOLR - B)])
    zeros_n = jnp.zeros((NPAD, H), f32)

    W_e1p = jnp.pad(W_e1, ((0, 8 - W_e1.shape[0]), (0, 0)))
    iota_c = jnp.arange(H * H, dtype=jnp.int32)
    Rm = (iota_c[None, :] // H == jnp.arange(H, dtype=jnp.int32)[:, None]
          ).astype(f32)
    Sm = (iota_c[:, None] % H == jnp.arange(H, dtype=jnp.int32)[None, :]
          ).astype(f32)
    B2m = b_e2.reshape(H, H)
    W_pe8 = jnp.pad(W_pe, ((0, 8 - W_pe.shape[0]), (0, 0)))
    b_e1r = b_e1.reshape(1, -1)
    b_e2r = b_e2.reshape(1, -1)
    b_per = b_pe.reshape(1, -1)
    b_projr = b_proj.reshape(1, -1)
    Wsplits = (W_ih[:, 0:H], W_ih[:, H:2 * H], W_ih[:, 2 * H:],
               W_hh[:, 0:H], W_hh[:, H:2 * H], W_hh[:, 2 * H:])
    b_ihr = b_ih.reshape(1, -1)
    b_hhr = b_hh.reshape(1, -1)
    W_f1a = W_f1[0:H]
    W_f1b = W_f1[H:2 * H]
    W_f1c = W_f1[2 * H:]

    h = _tc_h0(nf, W_proj, b_projr)
    edge_emb = _tc_edge_emb(ef8, W_pe8, b_per)
    for _ in range(3):
        hs = _sc_gather(h, src3)
        msg = _tc_edge(ef8, hs, W_e1p, b_e1r, W_e2, Rm, Sm, B2m)
        agg2 = _sc_scatter(msg, dst3, zeros_n)
        h = _tc_gru(agg2[0], agg2[1], h, Wsplits, b_ihr, b_hhr)
    hs_f = _sc_gather(h, src3)
    gid3 = _sc_gid(dst3, ngi)
    molA2, molB2 = _sc_scatter_mol(hs_f, edge_emb, gid3, zeros_n)
    return _tc_final(molA2, molB2, fp_vector, W_fp, b_fp,
                     bn_gamma.reshape(1, -1), bn_beta.reshape(1, -1),
                     W_f1a, W_f1b, W_f1c, b_f1.reshape(1, -1),
                     W_f2, b_f2.reshape(1, -1), W_head, b_head.reshape(1, -1))


# speedup vs baseline: 2.0585x; 1.2888x over previous
import functools

import jax
import jax.numpy as jnp
from jax import lax
from jax.experimental import pallas as pl
from jax.experimental.pallas import tpu as pltpu
from jax.experimental.pallas import tpu_sc as plsc

N = 50000
E = 100000
B = 1024
H = 32

NPAD = 50176
EPAD = 102400
NW = 32
NCHUNK = 25
CW = 128
MOLR = 1280

_SC_PARAMS = pltpu.CompilerParams(use_tc_tiling_on_sc=False)

TN = 400
TE = 512


def _h0_body(nf_ref, w_ref, b_ref, o_ref):
    o_ref[...] = jax.nn.relu(
        jnp.dot(nf_ref[...], w_ref[...], preferred_element_type=jnp.float32)
        + b_ref[...])


def _tc_h0(nf, W_proj, b_proj):
    return pl.pallas_call(
        _h0_body,
        grid=(N // TN,),
        in_specs=[
            pl.BlockSpec((TN, nf.shape[1]), lambda i: (i, 0)),
            pl.BlockSpec((nf.shape[1], H), lambda i: (0, 0)),
            pl.BlockSpec((1, H), lambda i: (0, 0)),
        ],
        out_specs=pl.BlockSpec((TN, H), lambda i: (i, 0)),
        out_shape=jax.ShapeDtypeStruct((N, H), jnp.float32),
    )(nf, W_proj, b_proj)


def _edge_emb_body(ef_ref, w_ref, b_ref, o_ref):
    o_ref[...] = jax.nn.relu(
        jnp.dot(ef_ref[...], w_ref[...], preferred_element_type=jnp.float32)
        + b_ref[...])


def _tc_edge_emb(ef8, W_pe8, b_pe):
    return pl.pallas_call(
        _edge_emb_body,
        grid=(EPAD // TN,),
        in_specs=[
            pl.BlockSpec((TN, 8), lambda i: (i, 0)),
            pl.BlockSpec((8, 32), lambda i: (0, 0)),
            pl.BlockSpec((1, 32), lambda i: (0, 0)),
        ],
        out_specs=pl.BlockSpec((TN, 32), lambda i: (i, 0)),
        out_shape=jax.ShapeDtypeStruct((EPAD, 32), jnp.float32),
    )(ef8, W_pe8, b_pe)


def _edge_body(ef_ref, hs_ref, w1_ref, b1_ref, w2_ref, r_ref, s4_ref, b2m_ref,
               o_ref):
    e1 = jax.nn.relu(
        jnp.dot(ef_ref[...], w1_ref[...], preferred_element_type=jnp.float32)
        + b1_ref[...])
    A = jnp.dot(e1.astype(jnp.bfloat16), w2_ref[...],
                preferred_element_type=jnp.float32)
    hs = hs_ref[...]
    hrep = jnp.dot(hs.astype(jnp.bfloat16), r_ref[...],
                   preferred_element_type=jnp.float32)
    P = hrep * A
    acc = (P[:, 0:128] + P[:, 128:256] + P[:, 256:384] + P[:, 384:512]
           + P[:, 512:640] + P[:, 640:768] + P[:, 768:896] + P[:, 896:1024])
    o_ref[...] = (
        jnp.dot(acc, s4_ref[...], preferred_element_type=jnp.float32)
        + jnp.dot(hs, b2m_ref[...], preferred_element_type=jnp.float32))


def _tc_edge(ef8, hs, W_e1p, b_e1, W_e2b, Rm, S4m, B2m):
    return pl.pallas_call(
        _edge_body,
        grid=(EPAD // TE,),
        in_specs=[
            pl.BlockSpec((TE, 8), lambda i: (i, 0)),
            pl.BlockSpec((TE, H), lambda i: (i, 0)),
            pl.BlockSpec((8, 128), lambda i: (0, 0)),
            pl.BlockSpec((1, 128), lambda i: (0, 0)),
            pl.BlockSpec((128, H * H), lambda i: (0, 0)),
            pl.BlockSpec((H, H * H), lambda i: (0, 0)),
            pl.BlockSpec((128, H), lambda i: (0, 0)),
            pl.BlockSpec((H, H), lambda i: (0, 0)),
        ],
        out_specs=pl.BlockSpec((TE, H), lambda i: (i, 0)),
        out_shape=jax.ShapeDtypeStruct((EPAD, H), jnp.float32),
    )(ef8, hs, W_e1p, b_e1, W_e2b, Rm, S4m, B2m)


def _gru_body(a0_ref, a1_ref, h_ref, wir, wiz, win, whr, whz, whn, bi, bh, o_ref):
    h = h_ref[...]
    m = jax.nn.relu(a0_ref[...] + a1_ref[...]) + h

    def mm(x, w):
        return jnp.dot(x, w[...], preferred_element_type=jnp.float32)

    r = jax.nn.sigmoid(mm(m, wir) + bi[:, 0:H] + mm(h, whr) + bh[:, 0:H])
    z = jax.nn.sigmoid(mm(m, wiz) + bi[:, H:2 * H] + mm(h, whz) + bh[:, H:2 * H])
    n = jnp.tanh(mm(m, win) + bi[:, 2 * H:] + r * (mm(h, whn) + bh[:, 2 * H:]))
    o_ref[...] = (1.0 - z) * n + z * h


def _tc_gru(a0, a1, h, Wsplits, b_ih, b_hh):
    wir, wiz, win, whr, whz, whn = Wsplits
    specs = [pl.BlockSpec((TN, H), lambda i: (i, 0))] * 3
    specs += [pl.BlockSpec((H, H), lambda i: (0, 0))] * 6
    specs += [pl.BlockSpec((1, 3 * H), lambda i: (0, 0))] * 2
    return pl.pallas_call(
        _gru_body,
        grid=(N // TN,),
        in_specs=specs,
        out_specs=pl.BlockSpec((TN, H), lambda i: (i, 0)),
        out_shape=jax.ShapeDtypeStruct((N, H), jnp.float32),
    )(a0, a1, h, wir, wiz, win, whr, whz, whn, b_ih, b_hh)


def _final_body(molA_ref, molB_ref, fp_ref, wfp_ref, bfp_ref, g_ref, be_ref,
                wf1a_ref, wf1b_ref, wf1c_ref, bf1_ref, wf2_ref, bf2_ref,
                wh_ref, bh_ref, o_ref):
    a = molA_ref[0, 0:B, :] + molA_ref[1, 0:B, :]
    b = molB_ref[0, 0:B, :] + molB_ref[1, 0:B, :]
    m = jnp.maximum(jnp.max(a, axis=1, keepdims=True),
                    jnp.max(b, axis=1, keepdims=True))
    ea = jnp.exp(a - m)
    eb = jnp.exp(b - m)
    s = jnp.sum(ea, axis=1, keepdims=True) + jnp.sum(eb, axis=1, keepdims=True)
    an = ea / s
    bn = eb / s
    x = jnp.dot(fp_ref[...], wfp_ref[...], preferred_element_type=jnp.float32) \
        + bfp_ref[...]
    mean = jnp.mean(x, axis=0, keepdims=True)
    var = jnp.mean(x * x, axis=0, keepdims=True) - mean * mean
    x = (x - mean) * jax.lax.rsqrt(var + 1e-5)
    x = jax.nn.relu(x * g_ref[...] + be_ref[...])

    def mm(p, w):
        return jnp.dot(p, w[...], preferred_element_type=jnp.float32)

    h1 = jax.nn.relu(mm(an, wf1a_ref) + mm(bn, wf1b_ref) + mm(x, wf1c_ref)
                     + bf1_ref[...])
    emb = mm(h1, wf2_ref) + bf2_ref[...]
    o_ref[...] = mm(emb, wh_ref) + bh_ref[...]


def _tc_final(molA2, molB2, fp, W_fp, b_fp, gam, bet,
              W_f1a, W_f1b, W_f1c, b_f1, W_f2, b_f2, W_head, b_head):
    args = (molA2, molB2, fp, W_fp, b_fp, gam, bet,
            W_f1a, W_f1b, W_f1c, b_f1, W_f2, b_f2, W_head, b_head)
    return pl.pallas_call(
        _final_body,
        in_specs=[pl.BlockSpec(a.shape, functools.partial(lambda r: (0,) * r, len(a.shape)))
                  for a in args],
        out_specs=pl.BlockSpec((B, W_head.shape[1]), lambda: (0, 0)),
        out_shape=jax.ShapeDtypeStruct((B, W_head.shape[1]), jnp.float32),
    )(*args)


def _sc_gather(h, idx3):
    mesh = plsc.VectorSubcoreMesh(core_axis_name="c", subcore_axis_name="s")

    @functools.partial(
        pl.kernel, mesh=mesh,
        out_type=jax.ShapeDtypeStruct((EPAD, H), jnp.float32),
        scratch_types=[
            pltpu.VMEM((NCHUNK, CW), jnp.int32),
            pltpu.VMEM((CW, H), jnp.float32),
            pltpu.SemaphoreType.DMA,
        ],
        compiler_params=_SC_PARAMS,
    )
    def k(h_hbm, idx_hbm, out_hbm, idx_v, rows_v, sem):
        c = lax.axis_index("c")
        s = lax.axis_index("s")
        wid = c * 16 + s
        pltpu.sync_copy(idx_hbm.at[wid], idx_v)

        def body(i, _):
            pltpu.async_copy(h_hbm.at[idx_v.at[i]], rows_v, sem).wait()
            pltpu.sync_copy(rows_v, out_hbm.at[pl.ds(wid * NCHUNK * CW + i * CW, CW)])
            return ()

        lax.fori_loop(0, NCHUNK, body, (), unroll=False)

    return k(h, idx3)


def _sc_scatter(msg, dst3, zeros_n):
    mesh = plsc.VectorSubcoreMesh(core_axis_name="c", subcore_axis_name="s")
    rows_per_tile = NPAD // 16

    @functools.partial(
        pl.kernel, mesh=mesh,
        out_type=jax.ShapeDtypeStruct((2, NPAD, H), jnp.float32),
        scratch_types=[
            pltpu.VMEM((NCHUNK, CW), jnp.int32),
            pltpu.VMEM((CW, H), jnp.float32),
            pltpu.VMEM_SHARED((NPAD, H), jnp.float32),
            pltpu.SemaphoreType.DMA,
        ],
        compiler_params=_SC_PARAMS,
    )
    def k(msg_hbm, idx_hbm, zero_hbm, out_hbm, idx_v, rows_v, acc, sem):
        c = lax.axis_index("c")
        s = lax.axis_index("s")
        wid = c * 16 + s
        pltpu.sync_copy(zero_hbm.at[pl.ds(0, rows_per_tile)],
                        acc.at[pl.ds(s * rows_per_tile, rows_per_tile)])
        pltpu.sync_copy(idx_hbm.at[wid], idx_v)
        plsc.subcore_barrier()

        def body(i, _):
            pltpu.async_copy(
                msg_hbm.at[pl.ds(wid * NCHUNK * CW + i * CW, CW)], rows_v, sem
            ).wait()
            pltpu.sync_copy(rows_v, acc.at[idx_v.at[i]], add=True)
            return ()

        lax.fori_loop(0, NCHUNK, body, (), unroll=False)
        plsc.subcore_barrier()
        pltpu.sync_copy(acc.at[pl.ds(s * rows_per_tile, rows_per_tile)],
                        out_hbm.at[c].at[pl.ds(s * rows_per_tile, rows_per_tile)])

    return k(msg, dst3, zeros_n)


def _sc_gid(dst3, ngi):
    mesh = plsc.VectorSubcoreMesh(core_axis_name="c", subcore_axis_name="s")
    rows_per_tile = NPAD // 16

    @functools.partial(
        pl.kernel, mesh=mesh,
        out_type=jax.ShapeDtypeStruct((NW, NCHUNK, CW), jnp.int32),
        scratch_types=[
            pltpu.VMEM((NCHUNK, CW), jnp.int32),
            pltpu.VMEM((NCHUNK, CW), jnp.int32),
            pltpu.VMEM((NPAD // 16,), jnp.int32),
            pltpu.VMEM_SHARED((NPAD,), jnp.int32),
            pltpu.SemaphoreType.DMA,
        ],
        compiler_params=_SC_PARAMS,
    )
    def k(dst_hbm, ngi_hbm, out_hbm, dst_v, gid_v, ngi_stage, ngi_sp, sem):
        c = lax.axis_index("c")
        s = lax.axis_index("s")
        wid = c * 16 + s
        pltpu.sync_copy(ngi_hbm.at[pl.ds(s * rows_per_tile, rows_per_tile)],
                        ngi_stage)
        pltpu.sync_copy(ngi_stage,
                        ngi_sp.at[pl.ds(s * rows_per_tile, rows_per_tile)])
        pltpu.sync_copy(dst_hbm.at[wid], dst_v)
        plsc.subcore_barrier()

        def body(i, _):
            pltpu.async_copy(ngi_sp.at[dst_v.at[i]], gid_v.at[i], sem).wait()
            return ()

        lax.fori_loop(0, NCHUNK, body, (), unroll=False)
        pltpu.sync_copy(gid_v, out_hbm.at[wid])

    return k(dst3, ngi)


def _sc_scatter_mol(msgA, msgB, gid3, zeros_n):
    mesh = plsc.VectorSubcoreMesh(core_axis_name="c", subcore_axis_name="s")
    mol_per_tile = MOLR // 16

    @functools.partial(
        pl.kernel, mesh=mesh,
        out_type=(jax.ShapeDtypeStruct((2, MOLR, H), jnp.float32),
                  jax.ShapeDtypeStruct((2, MOLR, H), jnp.float32)),
        scratch_types=[
            pltpu.VMEM((NCHUNK, CW), jnp.int32),
            pltpu.VMEM((CW, H), jnp.float32),
            pltpu.VMEM((CW, H), jnp.float32),
            pltpu.VMEM_SHARED((MOLR, H), jnp.float32),
            pltpu.VMEM_SHARED((MOLR, H), jnp.float32),
            pltpu.SemaphoreType.DMA,
        ],
        compiler_params=_SC_PARAMS,
    )
    def k(msgA_hbm, msgB_hbm, gid_hbm, zero_hbm, outA_hbm, outB_hbm,
          gid_v, arows_v, brows_v, accA, accB, sem):
        c = lax.axis_index("c")
        s = lax.axis_index("s")
        wid = c * 16 + s
        pltpu.sync_copy(zero_hbm.at[pl.ds(0, mol_per_tile)],
                        accA.at[pl.ds(s * mol_per_tile, mol_per_tile)])
        pltpu.sync_copy(zero_hbm.at[pl.ds(0, mol_per_tile)],
                        accB.at[pl.ds(s * mol_per_tile, mol_per_tile)])
        pltpu.sync_copy(gid_hbm.at[wid], gid_v)
        plsc.subcore_barrier()

        def body(i, _):
            base = wid * NCHUNK * CW + i * CW
            pltpu.async_copy(msgA_hbm.at[pl.ds(base, CW)], arows_v, sem).wait()
            pltpu.async_copy(msgB_hbm.at[pl.ds(base, CW)], brows_v, sem).wait()
            pltpu.sync_copy(arows_v, accA.at[gid_v.at[i]], add=True)
            pltpu.sync_copy(brows_v, accB.at[gid_v.at[i]], add=True)
            return ()

        lax.fori_loop(0, NCHUNK, body, (), unroll=False)
        plsc.subcore_barrier()
        pltpu.sync_copy(accA.at[pl.ds(s * mol_per_tile, mol_per_tile)],
                        outA_hbm.at[c].at[pl.ds(s * mol_per_tile, mol_per_tile)])
        pltpu.sync_copy(accB.at[pl.ds(s * mol_per_tile, mol_per_tile)],
                        outB_hbm.at[c].at[pl.ds(s * mol_per_tile, mol_per_tile)])

    return k(msgA, msgB, gid3, zeros_n)


def kernel(node_feats, edge_feats, fp_vector, edge_index, node_graph_ids,
           W_proj, b_proj, W_e1, b_e1, W_e2, b_e2,
           W_ih, W_hh, b_ih, b_hh, W_pe, b_pe,
           W_fp, b_fp, bn_gamma, bn_beta,
           W_f1, b_f1, W_f2, b_f2, W_head, b_head):
    f32 = jnp.float32
    ef8 = jnp.pad(edge_feats, ((0, EPAD - E), (0, 8 - edge_feats.shape[1])))
    pad_src = jnp.arange(EPAD - E, dtype=jnp.int32) % 128
    pad_dst = N + jnp.arange(EPAD - E, dtype=jnp.int32) % (NPAD - N)
    src = jnp.concatenate([edge_index[0].astype(jnp.int32), pad_src])
    dst = jnp.concatenate([edge_index[1].astype(jnp.int32), pad_dst])
    src3 = src.reshape(NW, NCHUNK, CW)
    dst3 = dst.reshape(NW, NCHUNK, CW)
    ngi = jnp.concatenate([
        node_graph_ids.astype(jnp.int32),
        B + jnp.arange(NPAD - N, dtype=jnp.int32) % (MOLR - B)])
    zeros_s = jnp.zeros((NPAD // 16, H), f32)

    W_e1p = jnp.pad(W_e1, ((0, 8 - W_e1.shape[0]), (0, 0)))
    iota_c = jnp.arange(H * H, dtype=jnp.int32)
    Rm = (iota_c[None, :] // H == jnp.arange(H, dtype=jnp.int32)[:, None]
          ).astype(jnp.bfloat16)
    S4m = (jnp.arange(128, dtype=jnp.int32)[:, None] % H
           == jnp.arange(H, dtype=jnp.int32)[None, :]).astype(f32)
    B2m = b_e2.reshape(H, H)
    W_e2b = W_e2.astype(jnp.bfloat16)
    W_pe8 = jnp.pad(W_pe, ((0, 8 - W_pe.shape[0]), (0, 0)))
    b_e1r = b_e1.reshape(1, -1)
    b_e2r = b_e2.reshape(1, -1)
    b_per = b_pe.reshape(1, -1)
    b_projr = b_proj.reshape(1, -1)
    Wsplits = (W_ih[:, 0:H], W_ih[:, H:2 * H], W_ih[:, 2 * H:],
               W_hh[:, 0:H], W_hh[:, H:2 * H], W_hh[:, 2 * H:])
    b_ihr = b_ih.reshape(1, -1)
    b_hhr = b_hh.reshape(1, -1)
    W_f1a = W_f1[0:H]
    W_f1b = W_f1[H:2 * H]
    W_f1c = W_f1[2 * H:]

    h = _tc_h0(node_feats, W_proj, b_projr)
    edge_emb = _tc_edge_emb(ef8, W_pe8, b_per)
    for _ in range(3):
        hs = _sc_gather(h, src3)
        msg = _tc_edge(ef8, hs, W_e1p, b_e1r, W_e2b, Rm, S4m, B2m)
        agg2 = _sc_scatter(msg, dst3, zeros_s)
        h = _tc_gru(agg2[0], agg2[1], h, Wsplits, b_ihr, b_hhr)
    hs_f = _sc_gather(h, src3)
    gid3 = _sc_gid(dst3, ngi)
    molA2, molB2 = _sc_scatter_mol(hs_f, edge_emb, gid3, zeros_s)
    return _tc_final(molA2, molB2, fp_vector, W_fp, b_fp,
                     bn_gamma.reshape(1, -1), bn_beta.reshape(1, -1),
                     W_f1a, W_f1b, W_f1c, b_f1.reshape(1, -1),
                     W_f2, b_f2.reshape(1, -1), W_head, b_head.reshape(1, -1))


# speedup vs baseline: 2.5516x; 1.2395x over previous
import functools

import jax
import jax.numpy as jnp
from jax import lax
from jax.experimental import pallas as pl
from jax.experimental.pallas import tpu as pltpu
from jax.experimental.pallas import tpu_sc as plsc

N = 50000
E = 100000
B = 1024
H = 32

NPAD = 50176
EPAD = 102400
NW = 32
NCHUNK = 25
CW = 128
MOLR = 1280

_SC_PARAMS = pltpu.CompilerParams(use_tc_tiling_on_sc=False)

TN = 2000
TE = 512


def _h0_body(nf_ref, w_ref, b_ref, o_ref):
    o_ref[...] = jax.nn.relu(
        jnp.dot(nf_ref[...], w_ref[...], preferred_element_type=jnp.float32)
        + b_ref[...])


def _tc_h0(nf, W_proj, b_proj):
    return pl.pallas_call(
        _h0_body,
        grid=(N // TN,),
        in_specs=[
            pl.BlockSpec((TN, nf.shape[1]), lambda i: (i, 0)),
            pl.BlockSpec((nf.shape[1], H), lambda i: (0, 0)),
            pl.BlockSpec((1, H), lambda i: (0, 0)),
        ],
        out_specs=pl.BlockSpec((TN, H), lambda i: (i, 0)),
        out_shape=jax.ShapeDtypeStruct((N, H), jnp.float32),
    )(nf, W_proj, b_proj)


def _edge_body(ef_ref, hs_ref, w1_ref, b1_ref, w2_ref, r_ref, s4_ref, b2m_ref,
               o_ref):
    e1 = jax.nn.relu(
        jnp.dot(ef_ref[...], w1_ref[...], preferred_element_type=jnp.float32)
        + b1_ref[...])
    A = jnp.dot(e1.astype(jnp.bfloat16), w2_ref[...],
                preferred_element_type=jnp.float32)
    hs = hs_ref[...]
    hrep = jnp.dot(hs.astype(jnp.bfloat16), r_ref[...],
                   preferred_element_type=jnp.float32)
    P = hrep * A
    acc = (P[:, 0:128] + P[:, 128:256] + P[:, 256:384] + P[:, 384:512]
           + P[:, 512:640] + P[:, 640:768] + P[:, 768:896] + P[:, 896:1024])
    o_ref[...] = (
        jnp.dot(acc, s4_ref[...], preferred_element_type=jnp.float32)
        + jnp.dot(hs, b2m_ref[...], preferred_element_type=jnp.float32))


def _edge_ee_body(ef_ref, hs_ref, w1_ref, b1_ref, w2_ref, r_ref, s4_ref,
                  b2m_ref, wpe_ref, bpe_ref, o_ref, ee_ref):
    _edge_body(ef_ref, hs_ref, w1_ref, b1_ref, w2_ref, r_ref, s4_ref, b2m_ref,
               o_ref)
    ee_ref[...] = jax.nn.relu(
        jnp.dot(ef_ref[...], wpe_ref[...], preferred_element_type=jnp.float32)
        + bpe_ref[...])


def _tc_edge(ef, hs, W_e1, b_e1, W_e2b, Rm, S4m, B2m, W_pe=None, b_pe=None):
    fin = W_pe is not None
    ncols = ef.shape[1]
    in_specs = [
        pl.BlockSpec((TE, ncols), lambda i: (i, 0)),
        pl.BlockSpec((TE, H), lambda i: (i, 0)),
        pl.BlockSpec((ncols, 128), lambda i: (0, 0)),
        pl.BlockSpec((1, 128), lambda i: (0, 0)),
        pl.BlockSpec((128, H * H), lambda i: (0, 0)),
        pl.BlockSpec((H, H * H), lambda i: (0, 0)),
        pl.BlockSpec((128, H), lambda i: (0, 0)),
        pl.BlockSpec((H, H), lambda i: (0, 0)),
    ]
    args = [ef, hs, W_e1, b_e1, W_e2b, Rm, S4m, B2m]
    out_specs = pl.BlockSpec((TE, H), lambda i: (i, 0))
    out_shape = jax.ShapeDtypeStruct((EPAD, H), jnp.float32)
    if fin:
        in_specs += [pl.BlockSpec((ncols, H), lambda i: (0, 0)),
                     pl.BlockSpec((1, H), lambda i: (0, 0))]
        args += [W_pe, b_pe]
        out_specs = (out_specs, pl.BlockSpec((TE, H), lambda i: (i, 0)))
        out_shape = (out_shape, jax.ShapeDtypeStruct((EPAD, H), jnp.float32))
    return pl.pallas_call(
        _edge_ee_body if fin else _edge_body,
        grid=(-(-E // TE),),
        in_specs=in_specs,
        out_specs=out_specs,
        out_shape=out_shape,
    )(*args)


def _gru_body(a0_ref, a1_ref, h_ref, wir, wiz, win, whr, whz, whn, bi, bh, o_ref):
    h = h_ref[...]
    m = jax.nn.relu(a0_ref[...] + a1_ref[...]) + h

    def mm(x, w):
        return jnp.dot(x, w[...], preferred_element_type=jnp.float32)

    r = jax.nn.sigmoid(mm(m, wir) + bi[:, 0:H] + mm(h, whr) + bh[:, 0:H])
    z = jax.nn.sigmoid(mm(m, wiz) + bi[:, H:2 * H] + mm(h, whz) + bh[:, H:2 * H])
    n = jnp.tanh(mm(m, win) + bi[:, 2 * H:] + r * (mm(h, whn) + bh[:, 2 * H:]))
    o_ref[...] = (1.0 - z) * n + z * h


def _tc_gru(a0, a1, h, Wsplits, b_ih, b_hh):
    wir, wiz, win, whr, whz, whn = Wsplits
    specs = [pl.BlockSpec((TN, H), lambda i: (i, 0))] * 3
    specs += [pl.BlockSpec((H, H), lambda i: (0, 0))] * 6
    specs += [pl.BlockSpec((1, 3 * H), lambda i: (0, 0))] * 2
    return pl.pallas_call(
        _gru_body,
        grid=(N // TN,),
        in_specs=specs,
        out_specs=pl.BlockSpec((TN, H), lambda i: (i, 0)),
        out_shape=jax.ShapeDtypeStruct((N, H), jnp.float32),
    )(a0, a1, h, wir, wiz, win, whr, whz, whn, b_ih, b_hh)


def _final_body(molA_ref, molB_ref, fp_ref, wfp_ref, bfp_ref, g_ref, be_ref,
                wf1a_ref, wf1b_ref, wf1c_ref, bf1_ref, wf2_ref, bf2_ref,
                wh_ref, bh_ref, o_ref):
    a = molA_ref[0, 0:B, :] + molA_ref[1, 0:B, :]
    b = molB_ref[0, 0:B, :] + molB_ref[1, 0:B, :]
    m = jnp.maximum(jnp.max(a, axis=1, keepdims=True),
                    jnp.max(b, axis=1, keepdims=True))
    ea = jnp.exp(a - m)
    eb = jnp.exp(b - m)
    s = jnp.sum(ea, axis=1, keepdims=True) + jnp.sum(eb, axis=1, keepdims=True)
    an = ea / s
    bn = eb / s
    x = jnp.dot(fp_ref[...], wfp_ref[...], preferred_element_type=jnp.float32) \
        + bfp_ref[...]
    mean = jnp.mean(x, axis=0, keepdims=True)
    var = jnp.mean(x * x, axis=0, keepdims=True) - mean * mean
    x = (x - mean) * jax.lax.rsqrt(var + 1e-5)
    x = jax.nn.relu(x * g_ref[...] + be_ref[...])

    def mm(p, w):
        return jnp.dot(p, w[...], preferred_element_type=jnp.float32)

    h1 = jax.nn.relu(mm(an, wf1a_ref) + mm(bn, wf1b_ref) + mm(x, wf1c_ref)
                     + bf1_ref[...])
    emb = mm(h1, wf2_ref) + bf2_ref[...]
    o_ref[...] = mm(emb, wh_ref) + bh_ref[...]


def _tc_final(molA2, molB2, fp, W_fp, b_fp, gam, bet,
              W_f1a, W_f1b, W_f1c, b_f1, W_f2, b_f2, W_head, b_head):
    args = (molA2, molB2, fp, W_fp, b_fp, gam, bet,
            W_f1a, W_f1b, W_f1c, b_f1, W_f2, b_f2, W_head, b_head)
    return pl.pallas_call(
        _final_body,
        in_specs=[pl.BlockSpec(a.shape, functools.partial(lambda r: (0,) * r, len(a.shape)))
                  for a in args],
        out_specs=pl.BlockSpec((B, W_head.shape[1]), lambda: (0, 0)),
        out_shape=jax.ShapeDtypeStruct((B, W_head.shape[1]), jnp.float32),
    )(*args)


def _sc_gather(h, idx3):
    mesh = plsc.VectorSubcoreMesh(core_axis_name="c", subcore_axis_name="s")

    @functools.partial(
        pl.kernel, mesh=mesh,
        out_type=jax.ShapeDtypeStruct((EPAD, H), jnp.float32),
        scratch_types=[
            pltpu.VMEM((NCHUNK, CW), jnp.int32),
            pltpu.VMEM((CW, H), jnp.float32),
            pltpu.SemaphoreType.DMA,
        ],
        compiler_params=_SC_PARAMS,
    )
    def k(h_hbm, idx_hbm, out_hbm, idx_v, rows_v, sem):
        c = lax.axis_index("c")
        s = lax.axis_index("s")
        wid = c * 16 + s
        pltpu.sync_copy(idx_hbm.at[wid], idx_v)

        def body(i, _):
            pltpu.async_copy(h_hbm.at[idx_v.at[i]], rows_v, sem).wait()
            pltpu.sync_copy(rows_v, out_hbm.at[pl.ds(wid * NCHUNK * CW + i * CW, CW)])
            return ()

        lax.fori_loop(0, NCHUNK, body, (), unroll=False)

    return k(h, idx3)


def _sc_scatter(msg, dst3, zeros_n):
    mesh = plsc.VectorSubcoreMesh(core_axis_name="c", subcore_axis_name="s")
    rows_per_tile = NPAD // 16

    @functools.partial(
        pl.kernel, mesh=mesh,
        out_type=jax.ShapeDtypeStruct((2, NPAD, H), jnp.float32),
        scratch_types=[
            pltpu.VMEM((NCHUNK, CW), jnp.int32),
            pltpu.VMEM((CW, H), jnp.float32),
            pltpu.VMEM_SHARED((NPAD, H), jnp.float32),
            pltpu.SemaphoreType.DMA,
        ],
        compiler_params=_SC_PARAMS,
    )
    def k(msg_hbm, idx_hbm, zero_hbm, out_hbm, idx_v, rows_v, acc, sem):
        c = lax.axis_index("c")
        s = lax.axis_index("s")
        wid = c * 16 + s
        pltpu.sync_copy(zero_hbm.at[pl.ds(0, rows_per_tile)],
                        acc.at[pl.ds(s * rows_per_tile, rows_per_tile)])
        pltpu.sync_copy(idx_hbm.at[wid], idx_v)
        plsc.subcore_barrier()

        def body(i, _):
            pltpu.async_copy(
                msg_hbm.at[pl.ds(wid * NCHUNK * CW + i * CW, CW)], rows_v, sem
            ).wait()
            pltpu.sync_copy(rows_v, acc.at[idx_v.at[i]], add=True)
            return ()

        lax.fori_loop(0, NCHUNK, body, (), unroll=False)
        plsc.subcore_barrier()
        pltpu.sync_copy(acc.at[pl.ds(s * rows_per_tile, rows_per_tile)],
                        out_hbm.at[c].at[pl.ds(s * rows_per_tile, rows_per_tile)])

    return k(msg, dst3, zeros_n)


def _sc_gid(dst3, ngi):
    mesh = plsc.VectorSubcoreMesh(core_axis_name="c", subcore_axis_name="s")
    rows_per_tile = NPAD // 16

    @functools.partial(
        pl.kernel, mesh=mesh,
        out_type=jax.ShapeDtypeStruct((NW, NCHUNK, CW), jnp.int32),
        scratch_types=[
            pltpu.VMEM((NCHUNK, CW), jnp.int32),
            pltpu.VMEM((NCHUNK, CW), jnp.int32),
            pltpu.VMEM((NPAD // 16,), jnp.int32),
            pltpu.VMEM_SHARED((NPAD,), jnp.int32),
            pltpu.SemaphoreType.DMA,
        ],
        compiler_params=_SC_PARAMS,
    )
    def k(dst_hbm, ngi_hbm, out_hbm, dst_v, gid_v, ngi_stage, ngi_sp, sem):
        c = lax.axis_index("c")
        s = lax.axis_index("s")
        wid = c * 16 + s
        pltpu.sync_copy(ngi_hbm.at[pl.ds(s * rows_per_tile, rows_per_tile)],
                        ngi_stage)
        pltpu.sync_copy(ngi_stage,
                        ngi_sp.at[pl.ds(s * rows_per_tile, rows_per_tile)])
        pltpu.sync_copy(dst_hbm.at[wid], dst_v)
        plsc.subcore_barrier()

        def body(i, _):
            pltpu.async_copy(ngi_sp.at[dst_v.at[i]], gid_v.at[i], sem).wait()
            return ()

        lax.fori_loop(0, NCHUNK, body, (), unroll=False)
        pltpu.sync_copy(gid_v, out_hbm.at[wid])

    return k(dst3, ngi)


def _sc_scatter_mol(msgA, msgB, gid3, zeros_n):
    mesh = plsc.VectorSubcoreMesh(core_axis_name="c", subcore_axis_name="s")
    mol_per_tile = MOLR // 16

    @functools.partial(
        pl.kernel, mesh=mesh,
        out_type=(jax.ShapeDtypeStruct((2, MOLR, H), jnp.float32),
                  jax.ShapeDtypeStruct((2, MOLR, H), jnp.float32)),
        scratch_types=[
            pltpu.VMEM((NCHUNK, CW), jnp.int32),
            pltpu.VMEM((CW, H), jnp.float32),
            pltpu.VMEM((CW, H), jnp.float32),
            pltpu.VMEM_SHARED((MOLR, H), jnp.float32),
            pltpu.VMEM_SHARED((MOLR, H), jnp.float32),
            pltpu.SemaphoreType.DMA,
        ],
        compiler_params=_SC_PARAMS,
    )
    def k(msgA_hbm, msgB_hbm, gid_hbm, zero_hbm, outA_hbm, outB_hbm,
          gid_v, arows_v, brows_v, accA, accB, sem):
        c = lax.axis_index("c")
        s = lax.axis_index("s")
        wid = c * 16 + s
        pltpu.sync_copy(zero_hbm.at[pl.ds(0, mol_per_tile)],
                        accA.at[pl.ds(s * mol_per_tile, mol_per_tile)])
        pltpu.sync_copy(zero_hbm.at[pl.ds(0, mol_per_tile)],
                        accB.at[pl.ds(s * mol_per_tile, mol_per_tile)])
        pltpu.sync_copy(gid_hbm.at[wid], gid_v)
        plsc.subcore_barrier()

        def body(i, _):
            base = wid * NCHUNK * CW + i * CW
            pltpu.async_copy(msgA_hbm.at[pl.ds(base, CW)], arows_v, sem).wait()
            pltpu.async_copy(msgB_hbm.at[pl.ds(base, CW)], brows_v, sem).wait()
            pltpu.sync_copy(arows_v, accA.at[gid_v.at[i]], add=True)
            pltpu.sync_copy(brows_v, accB.at[gid_v.at[i]], add=True)
            return ()

        lax.fori_loop(0, NCHUNK, body, (), unroll=False)
        plsc.subcore_barrier()
        pltpu.sync_copy(accA.at[pl.ds(s * mol_per_tile, mol_per_tile)],
                        outA_hbm.at[c].at[pl.ds(s * mol_per_tile, mol_per_tile)])
        pltpu.sync_copy(accB.at[pl.ds(s * mol_per_tile, mol_per_tile)],
                        outB_hbm.at[c].at[pl.ds(s * mol_per_tile, mol_per_tile)])

    return k(msgA, msgB, gid3, zeros_n)


def kernel(node_feats, edge_feats, fp_vector, edge_index, node_graph_ids,
           W_proj, b_proj, W_e1, b_e1, W_e2, b_e2,
           W_ih, W_hh, b_ih, b_hh, W_pe, b_pe,
           W_fp, b_fp, bn_gamma, bn_beta,
           W_f1, b_f1, W_f2, b_f2, W_head, b_head):
    f32 = jnp.float32
    pad_src = jnp.arange(EPAD - E, dtype=jnp.int32) % 128
    pad_dst = N + jnp.arange(EPAD - E, dtype=jnp.int32) % (NPAD - N)
    src = jnp.concatenate([edge_index[0].astype(jnp.int32), pad_src])
    dst = jnp.concatenate([edge_index[1].astype(jnp.int32), pad_dst])
    src3 = src.reshape(NW, NCHUNK, CW)
    dst3 = dst.reshape(NW, NCHUNK, CW)
    ngi = jnp.concatenate([
        node_graph_ids.astype(jnp.int32),
        B + jnp.arange(NPAD - N, dtype=jnp.int32) % (MOLR - B)])
    zeros_s = jnp.zeros((NPAD // 16, H), f32)

    iota_c = jnp.arange(H * H, dtype=jnp.int32)
    Rm = (iota_c[None, :] // H == jnp.arange(H, dtype=jnp.int32)[:, None]
          ).astype(jnp.bfloat16)
    S4m = (jnp.arange(128, dtype=jnp.int32)[:, None] % H
           == jnp.arange(H, dtype=jnp.int32)[None, :]).astype(f32)
    B2m = b_e2.reshape(H, H)
    W_e2b = W_e2.astype(jnp.bfloat16)
    b_e1r = b_e1.reshape(1, -1)
    b_e2r = b_e2.reshape(1, -1)
    b_per = b_pe.reshape(1, -1)
    b_projr = b_proj.reshape(1, -1)
    Wsplits = (W_ih[:, 0:H], W_ih[:, H:2 * H], W_ih[:, 2 * H:],
               W_hh[:, 0:H], W_hh[:, H:2 * H], W_hh[:, 2 * H:])
    b_ihr = b_ih.reshape(1, -1)
    b_hhr = b_hh.reshape(1, -1)
    W_f1a = W_f1[0:H]
    W_f1b = W_f1[H:2 * H]
    W_f1c = W_f1[2 * H:]

    h = _tc_h0(node_feats, W_proj, b_projr)
    edge_emb = None
    for step in range(3):
        hs = _sc_gather(h, src3)
        if step < 2:
            msg = _tc_edge(edge_feats, hs, W_e1, b_e1r, W_e2b, Rm, S4m, B2m)
        else:
            msg, edge_emb = _tc_edge(edge_feats, hs, W_e1, b_e1r, W_e2b, Rm,
                                     S4m, B2m, W_pe, b_per)
        agg2 = _sc_scatter(msg, dst3, zeros_s)
        h = _tc_gru(agg2[0], agg2[1], h, Wsplits, b_ihr, b_hhr)
    hs_f = _sc_gather(h, src3)
    gid3 = _sc_gid(dst3, ngi)
    molA2, molB2 = _sc_scatter_mol(hs_f, edge_emb, gid3, zeros_s)
    return _tc_final(molA2, molB2, fp_vector, W_fp, b_fp,
                     bn_gamma.reshape(1, -1), bn_beta.reshape(1, -1),
                     W_f1a, W_f1b, W_f1c, b_f1.reshape(1, -1),
                     W_f2, b_f2.reshape(1, -1), W_head, b_head.reshape(1, -1))


# speedup vs baseline: 2.5884x; 1.0145x over previous
import functools

import jax
import jax.numpy as jnp
from jax import lax
from jax.experimental import pallas as pl
from jax.experimental.pallas import tpu as pltpu
from jax.experimental.pallas import tpu_sc as plsc

N = 50000
E = 100000
B = 1024
H = 32

NPAD = 50176
EPAD = 102400
NW = 32
NCHUNK = 25
CW = 128
MOLR = 1280

_SC_PARAMS = pltpu.CompilerParams(use_tc_tiling_on_sc=False)

TN = 2000
TE = 512


def _h0_body(nf_ref, w_ref, b_ref, o_ref):
    o_ref[...] = jax.nn.relu(
        jnp.dot(nf_ref[...], w_ref[...], preferred_element_type=jnp.float32)
        + b_ref[...])


def _tc_h0(nf, W_proj, b_proj):
    return pl.pallas_call(
        _h0_body,
        grid=(N // TN,),
        in_specs=[
            pl.BlockSpec((TN, nf.shape[1]), lambda i: (i, 0)),
            pl.BlockSpec((nf.shape[1], H), lambda i: (0, 0)),
            pl.BlockSpec((1, H), lambda i: (0, 0)),
        ],
        out_specs=pl.BlockSpec((TN, H), lambda i: (i, 0)),
        out_shape=jax.ShapeDtypeStruct((N, H), jnp.float32),
    )(nf, W_proj, b_proj)


def _edge_body(ef_ref, hs_ref, w1_ref, b1_ref, w2_ref, r_ref, s4_ref, b2m_ref,
               o_ref):
    e1 = jax.nn.relu(
        jnp.dot(ef_ref[...], w1_ref[...], preferred_element_type=jnp.float32)
        + b1_ref[...])
    A = jnp.dot(e1.astype(jnp.bfloat16), w2_ref[...],
                preferred_element_type=jnp.float32)
    hs = hs_ref[...]
    hrep = jnp.dot(hs.astype(jnp.bfloat16), r_ref[...],
                   preferred_element_type=jnp.float32)
    P = hrep * A
    acc = (P[:, 0:128] + P[:, 128:256] + P[:, 256:384] + P[:, 384:512]
           + P[:, 512:640] + P[:, 640:768] + P[:, 768:896] + P[:, 896:1024])
    o_ref[...] = (
        jnp.dot(acc, s4_ref[...], preferred_element_type=jnp.float32)
        + jnp.dot(hs, b2m_ref[...], preferred_element_type=jnp.float32))


def _edge_ee_body(ef_ref, hs_ref, w1_ref, b1_ref, w2_ref, r_ref, s4_ref,
                  b2m_ref, wpe_ref, bpe_ref, o_ref, ee_ref):
    _edge_body(ef_ref, hs_ref, w1_ref, b1_ref, w2_ref, r_ref, s4_ref, b2m_ref,
               o_ref)
    ee_ref[...] = jax.nn.relu(
        jnp.dot(ef_ref[...], wpe_ref[...], preferred_element_type=jnp.float32)
        + bpe_ref[...])


def _tc_edge(ef, hs, W_e1, b_e1, W_e2b, Rm, S4m, B2m, W_pe=None, b_pe=None):
    fin = W_pe is not None
    ncols = ef.shape[1]
    in_specs = [
        pl.BlockSpec((TE, ncols), lambda i: (i, 0)),
        pl.BlockSpec((TE, H), lambda i: (i, 0)),
        pl.BlockSpec((ncols, 128), lambda i: (0, 0)),
        pl.BlockSpec((1, 128), lambda i: (0, 0)),
        pl.BlockSpec((128, H * H), lambda i: (0, 0)),
        pl.BlockSpec((H, H * H), lambda i: (0, 0)),
        pl.BlockSpec((128, H), lambda i: (0, 0)),
        pl.BlockSpec((H, H), lambda i: (0, 0)),
    ]
    args = [ef, hs, W_e1, b_e1, W_e2b, Rm, S4m, B2m]
    out_specs = pl.BlockSpec((TE, H), lambda i: (i, 0))
    out_shape = jax.ShapeDtypeStruct((EPAD, H), jnp.float32)
    if fin:
        in_specs += [pl.BlockSpec((ncols, H), lambda i: (0, 0)),
                     pl.BlockSpec((1, H), lambda i: (0, 0))]
        args += [W_pe, b_pe]
        out_specs = (out_specs, pl.BlockSpec((TE, H), lambda i: (i, 0)))
        out_shape = (out_shape, jax.ShapeDtypeStruct((EPAD, H), jnp.float32))
    return pl.pallas_call(
        _edge_ee_body if fin else _edge_body,
        grid=(-(-E // TE),),
        in_specs=in_specs,
        out_specs=out_specs,
        out_shape=out_shape,
    )(*args)


def _gru_body(a0_ref, a1_ref, h_ref, wir, wiz, win, whr, whz, whn, bi, bh, o_ref):
    h = h_ref[...]
    m = jax.nn.relu(a0_ref[...] + a1_ref[...]) + h

    def mm(x, w):
        return jnp.dot(x, w[...], preferred_element_type=jnp.float32)

    r = jax.nn.sigmoid(mm(m, wir) + bi[:, 0:H] + mm(h, whr) + bh[:, 0:H])
    z = jax.nn.sigmoid(mm(m, wiz) + bi[:, H:2 * H] + mm(h, whz) + bh[:, H:2 * H])
    n = jnp.tanh(mm(m, win) + bi[:, 2 * H:] + r * (mm(h, whn) + bh[:, 2 * H:]))
    o_ref[...] = (1.0 - z) * n + z * h


def _tc_gru(a0, a1, h, Wsplits, b_ih, b_hh):
    wir, wiz, win, whr, whz, whn = Wsplits
    specs = [pl.BlockSpec((TN, H), lambda i: (i, 0))] * 3
    specs += [pl.BlockSpec((H, H), lambda i: (0, 0))] * 6
    specs += [pl.BlockSpec((1, 3 * H), lambda i: (0, 0))] * 2
    return pl.pallas_call(
        _gru_body,
        grid=(N // TN,),
        in_specs=specs,
        out_specs=pl.BlockSpec((TN, H), lambda i: (i, 0)),
        out_shape=jax.ShapeDtypeStruct((N, H), jnp.float32),
    )(a0, a1, h, wir, wiz, win, whr, whz, whn, b_ih, b_hh)


def _final_body(molA_ref, molB_ref, fp_ref, wfp_ref, bfp_ref, g_ref, be_ref,
                wf1a_ref, wf1b_ref, wf1c_ref, bf1_ref, wf2_ref, bf2_ref,
                wh_ref, bh_ref, o_ref):
    a = molA_ref[0, 0:B, :] + molA_ref[1, 0:B, :]
    b = molB_ref[0, 0:B, :] + molB_ref[1, 0:B, :]
    m = jnp.maximum(jnp.max(a, axis=1, keepdims=True),
                    jnp.max(b, axis=1, keepdims=True))
    ea = jnp.exp(a - m)
    eb = jnp.exp(b - m)
    s = jnp.sum(ea, axis=1, keepdims=True) + jnp.sum(eb, axis=1, keepdims=True)
    an = ea / s
    bn = eb / s
    x = jnp.dot(fp_ref[...], wfp_ref[...], preferred_element_type=jnp.float32) \
        + bfp_ref[...]
    mean = jnp.mean(x, axis=0, keepdims=True)
    var = jnp.mean(x * x, axis=0, keepdims=True) - mean * mean
    x = (x - mean) * jax.lax.rsqrt(var + 1e-5)
    x = jax.nn.relu(x * g_ref[...] + be_ref[...])

    def mm(p, w):
        return jnp.dot(p, w[...], preferred_element_type=jnp.float32)

    h1 = jax.nn.relu(mm(an, wf1a_ref) + mm(bn, wf1b_ref) + mm(x, wf1c_ref)
                     + bf1_ref[...])
    emb = mm(h1, wf2_ref) + bf2_ref[...]
    o_ref[...] = mm(emb, wh_ref) + bh_ref[...]


def _tc_final(molA2, molB2, fp, W_fp, b_fp, gam, bet,
              W_f1a, W_f1b, W_f1c, b_f1, W_f2, b_f2, W_head, b_head):
    args = (molA2, molB2, fp, W_fp, b_fp, gam, bet,
            W_f1a, W_f1b, W_f1c, b_f1, W_f2, b_f2, W_head, b_head)
    return pl.pallas_call(
        _final_body,
        in_specs=[pl.BlockSpec(a.shape, functools.partial(lambda r: (0,) * r, len(a.shape)))
                  for a in args],
        out_specs=pl.BlockSpec((B, W_head.shape[1]), lambda: (0, 0)),
        out_shape=jax.ShapeDtypeStruct((B, W_head.shape[1]), jnp.float32),
    )(*args)


def _sc_gather(h, idx3):
    mesh = plsc.VectorSubcoreMesh(core_axis_name="c", subcore_axis_name="s")

    @functools.partial(
        pl.kernel, mesh=mesh,
        out_type=jax.ShapeDtypeStruct((EPAD, H), jnp.float32),
        scratch_types=[
            pltpu.VMEM((NCHUNK, CW), jnp.int32),
            pltpu.VMEM((CW, H), jnp.float32),
            pltpu.SemaphoreType.DMA,
        ],
        compiler_params=_SC_PARAMS,
    )
    def k(h_hbm, idx_hbm, out_hbm, idx_v, rows_v, sem):
        c = lax.axis_index("c")
        s = lax.axis_index("s")
        wid = c * 16 + s
        pltpu.sync_copy(idx_hbm.at[wid], idx_v)

        def body(i, _):
            pltpu.async_copy(h_hbm.at[idx_v.at[i]], rows_v, sem).wait()
            pltpu.sync_copy(rows_v, out_hbm.at[pl.ds(wid * NCHUNK * CW + i * CW, CW)])
            return ()

        lax.fori_loop(0, NCHUNK, body, (), unroll=False)

    return k(h, idx3)


def _sc_scatter(msg, dst3, zeros_n):
    mesh = plsc.VectorSubcoreMesh(core_axis_name="c", subcore_axis_name="s")
    rows_per_tile = NPAD // 16

    @functools.partial(
        pl.kernel, mesh=mesh,
        out_type=jax.ShapeDtypeStruct((2, NPAD, H), jnp.float32),
        scratch_types=[
            pltpu.VMEM((NCHUNK, CW), jnp.int32),
            pltpu.VMEM((CW, H), jnp.float32),
            pltpu.VMEM_SHARED((NPAD, H), jnp.float32),
            pltpu.SemaphoreType.DMA,
        ],
        compiler_params=_SC_PARAMS,
    )
    def k(msg_hbm, idx_hbm, zero_hbm, out_hbm, idx_v, rows_v, acc, sem):
        c = lax.axis_index("c")
        s = lax.axis_index("s")
        wid = c * 16 + s
        pltpu.sync_copy(zero_hbm.at[pl.ds(0, rows_per_tile)],
                        acc.at[pl.ds(s * rows_per_tile, rows_per_tile)])
        pltpu.sync_copy(idx_hbm.at[wid], idx_v)
        plsc.subcore_barrier()

        def body(i, _):
            pltpu.async_copy(
                msg_hbm.at[pl.ds(wid * NCHUNK * CW + i * CW, CW)], rows_v, sem
            ).wait()
            pltpu.sync_copy(rows_v, acc.at[idx_v.at[i]], add=True)
            return ()

        lax.fori_loop(0, NCHUNK, body, (), unroll=False)
        plsc.subcore_barrier()
        pltpu.sync_copy(acc.at[pl.ds(s * rows_per_tile, rows_per_tile)],
                        out_hbm.at[c].at[pl.ds(s * rows_per_tile, rows_per_tile)])

    return k(msg, dst3, zeros_n)


def _sc_readout(h3, src3, dst3, ee, ngi, zeros_s):
    mesh = plsc.VectorSubcoreMesh(core_axis_name="c", subcore_axis_name="s")
    mol_per_tile = MOLR // 16
    rows_per_tile = NPAD // 16

    @functools.partial(
        pl.kernel, mesh=mesh,
        out_type=(jax.ShapeDtypeStruct((2, MOLR, H), jnp.float32),
                  jax.ShapeDtypeStruct((2, MOLR, H), jnp.float32)),
        scratch_types=[
            pltpu.VMEM((NCHUNK, CW), jnp.int32),
            pltpu.VMEM((NCHUNK, CW), jnp.int32),
            pltpu.VMEM((CW,), jnp.int32),
            pltpu.VMEM((CW, H), jnp.float32),
            pltpu.VMEM((CW, H), jnp.float32),
            pltpu.VMEM((NPAD // 16,), jnp.int32),
            pltpu.VMEM_SHARED((NPAD,), jnp.int32),
            pltpu.VMEM_SHARED((MOLR, H), jnp.float32),
            pltpu.VMEM_SHARED((MOLR, H), jnp.float32),
            pltpu.SemaphoreType.DMA,
        ],
        compiler_params=_SC_PARAMS,
    )
    def k(h_hbm, src_hbm, dst_hbm, ee_hbm, ngi_hbm, zero_hbm,
          outA_hbm, outB_hbm,
          src_v, dst_v, gid_v, hrows_v, erows_v, ngi_stage, ngi_sp,
          accA, accB, sem):
        c = lax.axis_index("c")
        s = lax.axis_index("s")
        wid = c * 16 + s
        pltpu.sync_copy(zero_hbm.at[pl.ds(0, mol_per_tile)],
                        accA.at[pl.ds(s * mol_per_tile, mol_per_tile)])
        pltpu.sync_copy(zero_hbm.at[pl.ds(0, mol_per_tile)],
                        accB.at[pl.ds(s * mol_per_tile, mol_per_tile)])
        pltpu.sync_copy(ngi_hbm.at[pl.ds(s * rows_per_tile, rows_per_tile)],
                        ngi_stage)
        pltpu.sync_copy(ngi_stage,
                        ngi_sp.at[pl.ds(s * rows_per_tile, rows_per_tile)])
        pltpu.sync_copy(src_hbm.at[wid], src_v)
        pltpu.sync_copy(dst_hbm.at[wid], dst_v)
        plsc.subcore_barrier()

        def body(i, _):
            pltpu.async_copy(h_hbm.at[src_v.at[i]], hrows_v, sem).wait()
            pltpu.async_copy(ngi_sp.at[dst_v.at[i]], gid_v, sem).wait()
            pltpu.async_copy(
                ee_hbm.at[pl.ds(wid * NCHUNK * CW + i * CW, CW)], erows_v, sem
            ).wait()
            pltpu.sync_copy(hrows_v, accA.at[gid_v], add=True)
            pltpu.sync_copy(erows_v, accB.at[gid_v], add=True)
            return ()

        lax.fori_loop(0, NCHUNK, body, (), unroll=False)
        plsc.subcore_barrier()
        pltpu.sync_copy(accA.at[pl.ds(s * mol_per_tile, mol_per_tile)],
                        outA_hbm.at[c].at[pl.ds(s * mol_per_tile, mol_per_tile)])
        pltpu.sync_copy(accB.at[pl.ds(s * mol_per_tile, mol_per_tile)],
                        outB_hbm.at[c].at[pl.ds(s * mol_per_tile, mol_per_tile)])

    return k(h3, src3, dst3, ee, ngi, zeros_s)


def kernel(node_feats, edge_feats, fp_vector, edge_index, node_graph_ids,
           W_proj, b_proj, W_e1, b_e1, W_e2, b_e2,
           W_ih, W_hh, b_ih, b_hh, W_pe, b_pe,
           W_fp, b_fp, bn_gamma, bn_beta,
           W_f1, b_f1, W_f2, b_f2, W_head, b_head):
    f32 = jnp.float32
    pad_src = jnp.arange(EPAD - E, dtype=jnp.int32) % 128
    pad_dst = N + jnp.arange(EPAD - E, dtype=jnp.int32) % (NPAD - N)
    src = jnp.concatenate([edge_index[0].astype(jnp.int32), pad_src])
    dst = jnp.concatenate([edge_index[1].astype(jnp.int32), pad_dst])
    src3 = src.reshape(NW, NCHUNK, CW)
    dst3 = dst.reshape(NW, NCHUNK, CW)
    ngi = jnp.concatenate([
        node_graph_ids.astype(jnp.int32),
        B + jnp.arange(NPAD - N, dtype=jnp.int32) % (MOLR - B)])
    zeros_s = jnp.zeros((NPAD // 16, H), f32)

    iota_c = jnp.arange(H * H, dtype=jnp.int32)
    Rm = (iota_c[None, :] // H == jnp.arange(H, dtype=jnp.int32)[:, None]
          ).astype(jnp.bfloat16)
    S4m = (jnp.arange(128, dtype=jnp.int32)[:, None] % H
           == jnp.arange(H, dtype=jnp.int32)[None, :]).astype(f32)
    B2m = b_e2.reshape(H, H)
    W_e2b = W_e2.astype(jnp.bfloat16)
    b_e1r = b_e1.reshape(1, -1)
    b_e2r = b_e2.reshape(1, -1)
    b_per = b_pe.reshape(1, -1)
    b_projr = b_proj.reshape(1, -1)
    Wsplits = (W_ih[:, 0:H], W_ih[:, H:2 * H], W_ih[:, 2 * H:],
               W_hh[:, 0:H], W_hh[:, H:2 * H], W_hh[:, 2 * H:])
    b_ihr = b_ih.reshape(1, -1)
    b_hhr = b_hh.reshape(1, -1)
    W_f1a = W_f1[0:H]
    W_f1b = W_f1[H:2 * H]
    W_f1c = W_f1[2 * H:]

    h = _tc_h0(node_feats, W_proj, b_projr)
    edge_emb = None
    for step in range(3):
        hs = _sc_gather(h, src3)
        if step < 2:
            msg = _tc_edge(edge_feats, hs, W_e1, b_e1r, W_e2b, Rm, S4m, B2m)
        else:
            msg, edge_emb = _tc_edge(edge_feats, hs, W_e1, b_e1r, W_e2b, Rm,
                                     S4m, B2m, W_pe, b_per)
        agg2 = _sc_scatter(msg, dst3, zeros_s)
        h = _tc_gru(agg2[0], agg2[1], h, Wsplits, b_ihr, b_hhr)
    molA2, molB2 = _sc_readout(h, src3, dst3, edge_emb, ngi, zeros_s)
    return _tc_final(molA2, molB2, fp_vector, W_fp, b_fp,
                     bn_gamma.reshape(1, -1), bn_beta.reshape(1, -1),
                     W_f1a, W_f1b, W_f1c, b_f1.reshape(1, -1),
                     W_f2, b_f2.reshape(1, -1), W_head, b_head.reshape(1, -1))
